# baseline (speedup 1.0000x reference)
import functools

import jax
import jax.numpy as jnp
from jax import lax
from jax.experimental import pallas as pl
from jax.experimental.pallas import tpu as pltpu
from jax.experimental.pallas import tpu_sc as plsc

B, S, D = 1, 2048, 1024
E, FF = 64, 704
CAP = 40
THRESH = 0.2
T = B * S
EC = E * CAP
CH = 256
NCH = T // CH
NC, NS = 2, 16
NW = NC * NS
TPW = T // NW


def _router_body(x_ref, wrt_ref, ptr_ref, idx_ref, gatem_ref, perm_ref, nact_ref,
                 ol_ref, novf_ref):
    x = x_ref[...]
    logits = lax.dot_general(x, wrt_ref[...], (((1,), (1,)), ((), ())),
                             preferred_element_type=jnp.float32)
    m = jnp.max(logits, axis=-1, keepdims=True)
    s = jnp.sum(jnp.exp(logits - m), axis=-1, keepdims=True)
    top_val = 1.0 / s
    lane = lax.broadcasted_iota(jnp.int32, (T, E), 1)
    top_idx = jnp.min(jnp.where(logits == m, lane, E), axis=-1, keepdims=True)
    skip = top_val < THRESH
    gate = jnp.where(skip, 0.0, top_val)
    oh = (lane == top_idx).astype(jnp.float32)

    r = lax.broadcasted_iota(jnp.int32, (CH, CH), 0)
    c = lax.broadcasted_iota(jnp.int32, (CH, CH), 1)
    tril = (r >= c).astype(jnp.float32)
    acc = jnp.zeros((1, E), jnp.float32)
    pos_chunks = []
    for k in range(NCH):
        ohk = oh[k * CH:(k + 1) * CH, :]
        cs = jnp.dot(tril, ohk, preferred_element_type=jnp.float32) + acc
        pos_chunks.append(jnp.sum((cs - 1.0) * ohk, axis=-1, keepdims=True))
        acc = acc + jnp.sum(ohk, axis=0, keepdims=True)
    pos = jnp.concatenate(pos_chunks, axis=0)

    validf = jnp.where((pos < CAP) & (~skip), 1.0, 0.0)
    slotf = top_idx.astype(jnp.float32) * CAP + pos
    ptr = jnp.where(skip, -1,
                    jnp.where(validf > 0, slotf.astype(jnp.int32), EC))
    ptr_ref[...] = ptr

    slotv = jnp.where(validf > 0, slotf, -1.0)
    ovf = jnp.where((pos >= CAP) & (~skip), 1.0, 0.0)
    targets = lax.broadcasted_iota(jnp.int32, (1, EC), 1).astype(jnp.float32)
    otargets = lax.broadcasted_iota(jnp.int32, (1, T), 1).astype(jnp.float32)
    idxacc = jnp.zeros((1, EC), jnp.float32)
    gateacc = jnp.zeros((1, EC), jnp.float32)
    occacc = jnp.zeros((1, EC), jnp.float32)
    olacc = jnp.zeros((1, T), jnp.float32)
    oacc = jnp.zeros((1, 1), jnp.float32)
    for k in range(NCH):
        sk = slotv[k * CH:(k + 1) * CH, :]
        gk = gate[k * CH:(k + 1) * CH, :]
        tk = lax.broadcasted_iota(jnp.int32, (CH, 1), 0).astype(jnp.float32) + (k * CH)
        eq = sk == targets
        idxacc = idxacc + jnp.sum(jnp.where(eq, tk, 0.0), axis=0, keepdims=True)
        gateacc = gateacc + jnp.sum(jnp.where(eq, gk, 0.0), axis=0, keepdims=True)
        occacc = occacc + jnp.sum(jnp.where(eq, 1.0, 0.0), axis=0, keepdims=True)
        ok = ovf[k * CH:(k + 1) * CH, :]
        orank = jnp.dot(tril, ok, preferred_element_type=jnp.float32) + oacc
        eqo = (orank == otargets + 1.0) & (ok > 0)
        olacc = olacc + jnp.sum(jnp.where(eqo, tk, 0.0), axis=0, keepdims=True)
        oacc = oacc + jnp.sum(ok, axis=0, keepdims=True)
    idx_ref[...] = jnp.where(occacc > 0, idxacc, float(T)).astype(jnp.int32)
    gatem_ref[...] = gateacc
    ol_ref[...] = olacc.astype(jnp.int32)
    novf_ref[...] = oacc.astype(jnp.int32)

    counts = jnp.sum(oh * validf, axis=0, keepdims=True)
    activef = jnp.where(counts > 0, 1.0, 0.0)
    er = lax.broadcasted_iota(jnp.int32, (E, E), 0)
    ec = lax.broadcasted_iota(jnp.int32, (E, E), 1)
    upper = (er <= ec).astype(jnp.float32)
    rank = jnp.dot(activef, upper, preferred_element_type=jnp.float32)
    nact = jnp.sum(activef, axis=-1, keepdims=True)
    eye = (er == ec).astype(jnp.float32)
    rank_col = jnp.sum(jnp.broadcast_to(rank, (E, E)) * eye, axis=-1, keepdims=True)
    act_col = jnp.sum(jnp.broadcast_to(activef, (E, E)) * eye, axis=-1, keepdims=True)
    j_row = lax.broadcasted_iota(jnp.int32, (1, E), 1).astype(jnp.float32)
    e_col = lax.broadcasted_iota(jnp.int32, (E, 1), 0).astype(jnp.float32)
    hit = (rank_col == j_row + 1.0) & (act_col > 0)
    perm0 = jnp.sum(jnp.where(hit, e_col, 0.0), axis=0, keepdims=True)
    lasth = (rank_col == nact) & (act_col > 0)
    last = jnp.sum(jnp.where(lasth, e_col, 0.0))
    perm = jnp.where(j_row < nact, perm0, last).astype(jnp.int32)
    perm_ref[...] = perm
    nact_ref[...] = nact.astype(jnp.int32)


def _expert_body(perm_s, nact_s, idx_s, ol_s, novf_s,
                 x_hbm, gatem_ref, wg_ref, wu_ref, wd_ref, scout_hbm,
                 out_hbm, xe_s, ye_s, zr_s, sem_r, sem_o):
    del scout_hbm
    i = pl.program_id(0)
    n = nact_s[0]

    @pl.when(i == 0)
    def _():
        zr_s[...] = jnp.zeros_like(zr_s)

        def zbody(k, carry):
            t = ol_s[k]
            co = pltpu.make_async_copy(zr_s.at[0], out_hbm.at[t], sem_o)
            co.start()
            co.wait()
            return carry

        lax.fori_loop(0, novf_s[0], zbody, 0)

    @pl.when(i < n)
    def _():
        e = perm_s[i]
        copies = []
        for cc in range(CAP):
            t = idx_s[e * CAP + cc]
            tg = jnp.where(t < T, t, 0)
            cp = pltpu.make_async_copy(x_hbm.at[tg], xe_s.at[cc], sem_r)
            cp.start()
            copies.append(cp)
        for cp in copies:
            cp.wait()
        xe = xe_s[...]
        g = lax.dot_general(xe, wg_ref[0], (((1,), (1,)), ((), ())),
                            preferred_element_type=jnp.float32)
        u = lax.dot_general(xe, wu_ref[0], (((1,), (1,)), ((), ())),
                            preferred_element_type=jnp.float32)
        h = g * jax.nn.sigmoid(g) * u
        ye = jnp.dot(h, wd_ref[0], preferred_element_type=jnp.float32)
        gcol = gatem_ref[pl.ds(e * CAP, CAP), :]
        ye_s[...] = ye * gcol
        for cc in range(CAP):
            t = idx_s[e * CAP + cc]

            @pl.when(t < T)
            def _():
                co = pltpu.make_async_copy(ye_s.at[cc], out_hbm.at[t], sem_o)
                co.start()
                co.wait()


def _make_sc_copy():
    mesh = plsc.VectorSubcoreMesh(core_axis_name="c", subcore_axis_name="s",
                                  num_cores=NC, num_subcores=NS)

    @functools.partial(
        pl.kernel,
        out_type=jax.ShapeDtypeStruct((T, D), jnp.float32),
        mesh=mesh,
        scratch_types=[pltpu.VMEM((TPW, D), jnp.float32)],
    )
    def sccopy(x_hbm, out_hbm, chunk):
        wid = lax.axis_index("c") * NS + lax.axis_index("s")
        base = wid * TPW
        pltpu.sync_copy(x_hbm.at[pl.ds(base, TPW)], chunk)
        pltpu.sync_copy(chunk, out_hbm.at[pl.ds(base, TPW)])

    return sccopy


def kernel(hidden_states, Wr, Wg, Wu, Wd):
    x = hidden_states.reshape(T, D)

    ptr, idxm, gatem, perm, nact, ol, novf = pl.pallas_call(
        _router_body,
        out_shape=(
            jax.ShapeDtypeStruct((T, 1), jnp.int32),
            jax.ShapeDtypeStruct((1, EC), jnp.int32),
            jax.ShapeDtypeStruct((1, EC), jnp.float32),
            jax.ShapeDtypeStruct((1, E), jnp.int32),
            jax.ShapeDtypeStruct((1, 1), jnp.int32),
            jax.ShapeDtypeStruct((1, T), jnp.int32),
            jax.ShapeDtypeStruct((1, 1), jnp.int32),
        ),
    )(x, Wr.T)
    del ptr

    scout = _make_sc_copy()(x)

    out = pl.pallas_call(
        _expert_body,
        grid_spec=pltpu.PrefetchScalarGridSpec(
            num_scalar_prefetch=5,
            grid=(E,),
            in_specs=[
                pl.BlockSpec(memory_space=pltpu.MemorySpace.HBM),
                pl.BlockSpec((EC, 1), lambda i, *s: (0, 0)),
                pl.BlockSpec((1, FF, D), lambda i, *s: (s[0][i], 0, 0)),
                pl.BlockSpec((1, FF, D), lambda i, *s: (s[0][i], 0, 0)),
                pl.BlockSpec((1, FF, D), lambda i, *s: (s[0][i], 0, 0)),
                pl.BlockSpec(memory_space=pltpu.MemorySpace.HBM),
            ],
            out_specs=pl.BlockSpec(memory_space=pltpu.MemorySpace.HBM),
            scratch_shapes=[
                pltpu.VMEM((CAP, D), jnp.float32),
                pltpu.VMEM((CAP, D), jnp.float32),
                pltpu.VMEM((8, D), jnp.float32),
                pltpu.SemaphoreType.DMA,
                pltpu.SemaphoreType.DMA,
            ],
        ),
        out_shape=jax.ShapeDtypeStruct((T, D), jnp.float32),
        input_output_aliases={10: 0},
        compiler_params=pltpu.CompilerParams(
            dimension_semantics=("arbitrary",)),
    )(perm.reshape(E), nact.reshape(1), idxm.reshape(EC),
      ol.reshape(T), novf.reshape(1),
      x, gatem.reshape(EC, 1),
      jnp.swapaxes(Wg, 1, 2), jnp.swapaxes(Wu, 1, 2), Wd, scout)

    return out.reshape(B, S, D)


# speedup vs baseline: 1.3021x; 1.3021x over previous
import jax
import jax.numpy as jnp
from jax import lax
from jax.experimental import pallas as pl
from jax.experimental.pallas import tpu as pltpu

B, S, D = 1, 2048, 1024
E, FF = 64, 704
CAP = 40
THRESH = 0.2
T = B * S
EC = E * CAP
CH = 256
NCH = T // CH


def _router_body(x_ref, wrt_ref, ptr_ref, idx_ref, gatem_ref, perm_ref, nact_ref):
    x = x_ref[...]
    logits = lax.dot_general(x, wrt_ref[...], (((1,), (1,)), ((), ())),
                             preferred_element_type=jnp.float32)
    m = jnp.max(logits, axis=-1, keepdims=True)
    s = jnp.sum(jnp.exp(logits - m), axis=-1, keepdims=True)
    top_val = 1.0 / s
    lane = lax.broadcasted_iota(jnp.int32, (T, E), 1)
    top_idx = jnp.min(jnp.where(logits == m, lane, E), axis=-1, keepdims=True)
    skip = top_val < THRESH
    gate = jnp.where(skip, 0.0, top_val)
    oh = (lane == top_idx).astype(jnp.float32)

    r = lax.broadcasted_iota(jnp.int32, (CH, CH), 0)
    c = lax.broadcasted_iota(jnp.int32, (CH, CH), 1)
    tril = (r >= c).astype(jnp.float32)
    acc = jnp.zeros((1, E), jnp.float32)
    pos_chunks = []
    for k in range(NCH):
        ohk = oh[k * CH:(k + 1) * CH, :]
        cs = jnp.dot(tril, ohk, preferred_element_type=jnp.float32) + acc
        pos_chunks.append(jnp.sum((cs - 1.0) * ohk, axis=-1, keepdims=True))
        acc = acc + jnp.sum(ohk, axis=0, keepdims=True)
    pos = jnp.concatenate(pos_chunks, axis=0)

    validf = jnp.where((pos < CAP) & (~skip), 1.0, 0.0)
    slotf = top_idx.astype(jnp.float32) * CAP + pos
    ptr = jnp.where(skip, -1,
                    jnp.where(validf > 0, slotf.astype(jnp.int32), EC))
    ptr_ref[...] = ptr

    slotv = jnp.where(validf > 0, slotf, -1.0)
    targets = lax.broadcasted_iota(jnp.int32, (1, EC), 1).astype(jnp.float32)
    idxacc = jnp.zeros((1, EC), jnp.float32)
    gateacc = jnp.zeros((1, EC), jnp.float32)
    occacc = jnp.zeros((1, EC), jnp.float32)
    for k in range(NCH):
        sk = slotv[k * CH:(k + 1) * CH, :]
        gk = gate[k * CH:(k + 1) * CH, :]
        tk = lax.broadcasted_iota(jnp.int32, (CH, 1), 0).astype(jnp.float32) + (k * CH)
        eq = sk == targets
        idxacc = idxacc + jnp.sum(jnp.where(eq, tk, 0.0), axis=0, keepdims=True)
        gateacc = gateacc + jnp.sum(jnp.where(eq, gk, 0.0), axis=0, keepdims=True)
        occacc = occacc + jnp.sum(jnp.where(eq, 1.0, 0.0), axis=0, keepdims=True)
    idx_ref[...] = jnp.where(occacc > 0, idxacc, float(T)).astype(jnp.int32)
    gatem_ref[...] = gateacc

    counts = jnp.sum(oh * validf, axis=0, keepdims=True)
    activef = jnp.where(counts > 0, 1.0, 0.0)
    er = lax.broadcasted_iota(jnp.int32, (E, E), 0)
    ec = lax.broadcasted_iota(jnp.int32, (E, E), 1)
    upper = (er <= ec).astype(jnp.float32)
    rank = jnp.dot(activef, upper, preferred_element_type=jnp.float32)
    nact = jnp.sum(activef, axis=-1, keepdims=True)
    eye = (er == ec).astype(jnp.float32)
    rank_col = jnp.sum(jnp.broadcast_to(rank, (E, E)) * eye, axis=-1, keepdims=True)
    act_col = jnp.sum(jnp.broadcast_to(activef, (E, E)) * eye, axis=-1, keepdims=True)
    j_row = lax.broadcasted_iota(jnp.int32, (1, E), 1).astype(jnp.float32)
    e_col = lax.broadcasted_iota(jnp.int32, (E, 1), 0).astype(jnp.float32)
    hit = (rank_col == j_row + 1.0) & (act_col > 0)
    perm0 = jnp.sum(jnp.where(hit, e_col, 0.0), axis=0, keepdims=True)
    lasth = (rank_col == nact) & (act_col > 0)
    last = jnp.sum(jnp.where(lasth, e_col, 0.0))
    perm = jnp.where(j_row < nact, perm0, last).astype(jnp.int32)
    perm_ref[...] = perm
    nact_ref[...] = nact.astype(jnp.int32)


def _expert_body(perm_s, nact_s, idx_s,
                 x_ref, ptr_ref, gatem_ref, wg_ref, wu_ref, wd_ref,
                 out_ref, xe_s):
    i = pl.program_id(0)
    n = nact_s[0]

    @pl.when(i == 0)
    def _():
        out_ref[...] = jnp.where(ptr_ref[...] < 0, x_ref[...], 0.0)

    @pl.when(i < n)
    def _():
        e = perm_s[i]
        for cc in range(CAP):
            t = idx_s[e * CAP + cc]
            tg = jnp.where(t < T, t, 0)
            xe_s[cc:cc + 1, :] = x_ref[pl.ds(tg, 1), :]
        xe = xe_s[...]
        g = lax.dot_general(xe, wg_ref[0], (((1,), (1,)), ((), ())),
                            preferred_element_type=jnp.float32)
        u = lax.dot_general(xe, wu_ref[0], (((1,), (1,)), ((), ())),
                            preferred_element_type=jnp.float32)
        h = g * jax.nn.sigmoid(g) * u
        ye = jnp.dot(h, wd_ref[0], preferred_element_type=jnp.float32)
        gcol = gatem_ref[pl.ds(e * CAP, CAP), :]
        ye = ye * gcol
        for cc in range(CAP):
            t = idx_s[e * CAP + cc]

            @pl.when(t < T)
            def _():
                out_ref[pl.ds(t, 1), :] = ye[cc:cc + 1, :]


def kernel(hidden_states, Wr, Wg, Wu, Wd):
    x = hidden_states.reshape(T, D)

    ptr, idxm, gatem, perm, nact = pl.pallas_call(
        _router_body,
        out_shape=(
            jax.ShapeDtypeStruct((T, 1), jnp.int32),
            jax.ShapeDtypeStruct((1, EC), jnp.int32),
            jax.ShapeDtypeStruct((1, EC), jnp.float32),
            jax.ShapeDtypeStruct((1, E), jnp.int32),
            jax.ShapeDtypeStruct((1, 1), jnp.int32),
        ),
    )(x, Wr.T)

    out = pl.pallas_call(
        _expert_body,
        grid_spec=pltpu.PrefetchScalarGridSpec(
            num_scalar_prefetch=3,
            grid=(E,),
            in_specs=[
                pl.BlockSpec((T, D), lambda i, *s: (0, 0)),
                pl.BlockSpec((T, 1), lambda i, *s: (0, 0)),
                pl.BlockSpec((EC, 1), lambda i, *s: (0, 0)),
                pl.BlockSpec((1, FF, D), lambda i, p, n, ix: (p[i], 0, 0)),
                pl.BlockSpec((1, FF, D), lambda i, p, n, ix: (p[i], 0, 0)),
                pl.BlockSpec((1, FF, D), lambda i, p, n, ix: (p[i], 0, 0)),
            ],
            out_specs=pl.BlockSpec((T, D), lambda i, *s: (0, 0)),
            scratch_shapes=[pltpu.VMEM((CAP, D), jnp.float32)],
        ),
        out_shape=jax.ShapeDtypeStruct((T, D), jnp.float32),
        compiler_params=pltpu.CompilerParams(
            dimension_semantics=("arbitrary",)),
    )(perm.reshape(E), nact.reshape(1), idxm.reshape(EC),
      x, ptr, gatem.reshape(EC, 1),
      jnp.swapaxes(Wg, 1, 2), jnp.swapaxes(Wu, 1, 2), Wd)

    return out.reshape(B, S, D)
